# Initial kernel scaffold; baseline (speedup 1.0000x reference)
import jax
import jax.numpy as jnp
from jax.experimental import pallas as pl


def kernel(x, ctx_mod, context, W, b):
    raise NotImplementedError("write your pallas kernel here")


# speedup vs baseline: 2.7842x; 2.7842x over previous
import functools

import jax
import jax.numpy as jnp
from jax.experimental import pallas as pl


def _pre_kernel(ctx_ref, cm_ref, cn_ref, m_ref):
    c = ctx_ref[...]
    norms = jnp.sqrt(jnp.sum(c * c, axis=1, keepdims=True))
    cn_ref[...] = c / jnp.clip(norms, 1e-8, None)
    seg = jax.lax.dot_general(
        cm_ref[...], c, (((1,), (1,)), ((), ())),
        preferred_element_type=jnp.float32)
    m_ref[...] = jnp.max(seg, axis=0, keepdims=True)


def _main_kernel(x_ref, cnt_ref, m_ref, wt_ref, b_ref, out_ref, *, n_total):
    n = pl.program_id(1)
    xb = x_ref[0]
    s = jnp.dot(xb, cnt_ref[...], preferred_element_type=jnp.float32)
    n_ctx = s.shape[1]
    rowmax = jnp.max(s, axis=1, keepdims=True)
    idx = jax.lax.broadcasted_iota(jnp.int32, s.shape, 1)
    argm = jnp.min(jnp.where(s == rowmax, idx, n_ctx), axis=1, keepdims=True)
    mval = jnp.max(jnp.where(idx == argm, m_ref[...], -jnp.inf),
                   axis=1, keepdims=True)
    act = jax.nn.sigmoid(mval)
    h = jnp.dot(xb, wt_ref[...], preferred_element_type=jnp.float32) + b_ref[...]
    h = 0.5 * h * (1.0 + jax.lax.erf(h * 0.7071067811865476))
    contrib = h * act * (1.0 / n_total)

    @pl.when(n == 0)
    def _():
        out_ref[...] = contrib

    @pl.when(n > 0)
    def _():
        out_ref[...] += contrib


def kernel(x, ctx_mod, context, W, b):
    N, B, L = x.shape
    n_ctx = context.shape[0]

    cn, m = pl.pallas_call(
        _pre_kernel,
        out_shape=[
            jax.ShapeDtypeStruct((n_ctx, L), jnp.float32),
            jax.ShapeDtypeStruct((1, n_ctx), jnp.float32),
        ],
    )(context, ctx_mod)

    cnt = cn.T
    wt = W.T
    b2 = b.reshape(1, L)

    tb = 256 if B % 256 == 0 else B
    nb = B // tb
    out = pl.pallas_call(
        functools.partial(_main_kernel, n_total=N),
        grid=(nb, N),
        in_specs=[
            pl.BlockSpec((1, tb, L), lambda bi, n: (n, bi, 0)),
            pl.BlockSpec((L, n_ctx), lambda bi, n: (0, 0)),
            pl.BlockSpec((1, n_ctx), lambda bi, n: (0, 0)),
            pl.BlockSpec((L, L), lambda bi, n: (0, 0)),
            pl.BlockSpec((1, L), lambda bi, n: (0, 0)),
        ],
        out_specs=pl.BlockSpec((tb, L), lambda bi, n: (bi, 0)),
        out_shape=jax.ShapeDtypeStruct((B, L), jnp.float32),
    )(x, cnt, m, wt, b2)
    return out


# speedup vs baseline: 2.8709x; 1.0311x over previous
import functools

import jax
import jax.numpy as jnp
from jax.experimental import pallas as pl


def _pre_kernel(ctx_ref, cm_ref, cn_ref, m_ref):
    c = ctx_ref[...]
    norms = jnp.sqrt(jnp.sum(c * c, axis=1, keepdims=True))
    cn_ref[...] = c / jnp.clip(norms, 1e-8, None)
    seg = jax.lax.dot_general(
        cm_ref[...], c, (((1,), (1,)), ((), ())),
        preferred_element_type=jnp.float32)
    m_ref[...] = jnp.max(seg, axis=0, keepdims=True)


def _main_kernel(x_ref, cnt_ref, m_ref, wt_ref, b_ref, out_ref, *, n_total):
    n = pl.program_id(1)
    xb = x_ref[0]
    xb16 = xb.astype(jnp.bfloat16)
    s = jnp.dot(xb16, cnt_ref[...], preferred_element_type=jnp.float32)
    n_ctx = s.shape[1]
    rowmax = jnp.max(s, axis=1, keepdims=True)
    idx = jax.lax.broadcasted_iota(jnp.int32, s.shape, 1)
    argm = jnp.min(jnp.where(s == rowmax, idx, n_ctx), axis=1, keepdims=True)
    mval = jnp.max(jnp.where(idx == argm, m_ref[...], -jnp.inf),
                   axis=1, keepdims=True)
    act = jax.nn.sigmoid(mval)
    h = jnp.dot(xb16, wt_ref[...], preferred_element_type=jnp.float32) + b_ref[...]
    h = 0.5 * h * (1.0 + jax.lax.erf(h * 0.7071067811865476))
    contrib = h * act * (1.0 / n_total)

    @pl.when(n == 0)
    def _():
        out_ref[...] = contrib

    @pl.when(n > 0)
    def _():
        out_ref[...] += contrib


def kernel(x, ctx_mod, context, W, b):
    N, B, L = x.shape
    n_ctx = context.shape[0]

    cn, m = pl.pallas_call(
        _pre_kernel,
        out_shape=[
            jax.ShapeDtypeStruct((n_ctx, L), jnp.float32),
            jax.ShapeDtypeStruct((1, n_ctx), jnp.float32),
        ],
    )(context, ctx_mod)

    cnt = cn.T.astype(jnp.bfloat16)
    wt = W.T.astype(jnp.bfloat16)
    b2 = b.reshape(1, L)

    tb = 256 if B % 256 == 0 else B
    nb = B // tb
    out = pl.pallas_call(
        functools.partial(_main_kernel, n_total=N),
        grid=(nb, N),
        in_specs=[
            pl.BlockSpec((1, tb, L), lambda bi, n: (n, bi, 0)),
            pl.BlockSpec((L, n_ctx), lambda bi, n: (0, 0)),
            pl.BlockSpec((1, n_ctx), lambda bi, n: (0, 0)),
            pl.BlockSpec((L, L), lambda bi, n: (0, 0)),
            pl.BlockSpec((1, L), lambda bi, n: (0, 0)),
        ],
        out_specs=pl.BlockSpec((tb, L), lambda bi, n: (bi, 0)),
        out_shape=jax.ShapeDtypeStruct((B, L), jnp.float32),
    )(x, cnt, m, wt, b2)
    return out


# speedup vs baseline: 3.4167x; 1.1901x over previous
import functools

import jax
import jax.numpy as jnp
from jax.experimental import pallas as pl


def _pre_kernel(ctx_ref, cm_ref, cn_ref, m_ref):
    c = ctx_ref[...]
    norms = jnp.sqrt(jnp.sum(c * c, axis=1, keepdims=True))
    cn_ref[...] = c / jnp.clip(norms, 1e-8, None)
    seg = jax.lax.dot_general(
        cm_ref[...], c, (((1,), (1,)), ((), ())),
        preferred_element_type=jnp.float32)
    m_ref[...] = jnp.max(seg, axis=0, keepdims=True)


def _main_kernel(x_ref, cnt_ref, m_ref, wt_ref, b_ref, out_ref, *, n_total):
    n = pl.program_id(1)
    xb = x_ref[0]
    xb16 = xb.astype(jnp.bfloat16)
    s = jnp.dot(xb16, cnt_ref[...], preferred_element_type=jnp.float32)
    rowmax = jnp.max(s, axis=1, keepdims=True)
    mval = jnp.max(jnp.where(s == rowmax, m_ref[...], -jnp.inf),
                   axis=1, keepdims=True)
    act = jax.nn.sigmoid(mval)
    h = jnp.dot(xb16, wt_ref[...], preferred_element_type=jnp.float32) + b_ref[...]
    h = 0.5 * h * (1.0 + jax.lax.erf(h * 0.7071067811865476))
    contrib = h * act * (1.0 / n_total)

    @pl.when(n == 0)
    def _():
        out_ref[...] = contrib

    @pl.when(n > 0)
    def _():
        out_ref[...] += contrib


def kernel(x, ctx_mod, context, W, b):
    N, B, L = x.shape
    n_ctx = context.shape[0]

    cn, m = pl.pallas_call(
        _pre_kernel,
        out_shape=[
            jax.ShapeDtypeStruct((n_ctx, L), jnp.float32),
            jax.ShapeDtypeStruct((1, n_ctx), jnp.float32),
        ],
    )(context, ctx_mod)

    cnt = cn.T.astype(jnp.bfloat16)
    wt = W.T.astype(jnp.bfloat16)
    b2 = b.reshape(1, L)

    tb = 512 if B % 512 == 0 else B
    nb = B // tb
    out = pl.pallas_call(
        functools.partial(_main_kernel, n_total=N),
        grid=(nb, N),
        in_specs=[
            pl.BlockSpec((1, tb, L), lambda bi, n: (n, bi, 0)),
            pl.BlockSpec((L, n_ctx), lambda bi, n: (0, 0)),
            pl.BlockSpec((1, n_ctx), lambda bi, n: (0, 0)),
            pl.BlockSpec((L, L), lambda bi, n: (0, 0)),
            pl.BlockSpec((1, L), lambda bi, n: (0, 0)),
        ],
        out_specs=pl.BlockSpec((tb, L), lambda bi, n: (bi, 0)),
        out_shape=jax.ShapeDtypeStruct((B, L), jnp.float32),
    )(x, cnt, m, wt, b2)
    return out


# speedup vs baseline: 3.7176x; 1.0881x over previous
import functools

import jax
import jax.numpy as jnp
from jax.experimental import pallas as pl

_NT = (((1,), (1,)), ((), ()))


def _pre_kernel(ctx_ref, cm_ref, w_ref, cn_ref, m_ref, w16_ref):
    c = ctx_ref[...]
    norms = jnp.sqrt(jnp.sum(c * c, axis=1, keepdims=True))
    cn_ref[...] = (c / jnp.clip(norms, 1e-8, None)).astype(jnp.bfloat16)
    seg = jax.lax.dot_general(cm_ref[...], c, _NT,
                              preferred_element_type=jnp.float32)
    m_ref[...] = jnp.max(seg, axis=0, keepdims=True)
    w16_ref[...] = w_ref[...].astype(jnp.bfloat16)


def _main_kernel(x_ref, cn_ref, m_ref, w16_ref, b_ref, out_ref, *, n_total):
    n = pl.program_id(1)
    xb16 = x_ref[0].astype(jnp.bfloat16)
    s = jax.lax.dot_general(xb16, cn_ref[...], _NT,
                            preferred_element_type=jnp.float32)
    rowmax = jnp.max(s, axis=1, keepdims=True)
    mval = jnp.max(jnp.where(s == rowmax, m_ref[...], -jnp.inf),
                   axis=1, keepdims=True)
    act = jax.nn.sigmoid(mval) * (0.5 / n_total)
    h = jax.lax.dot_general(xb16, w16_ref[...], _NT,
                            preferred_element_type=jnp.float32) + b_ref[...]
    g = h * (1.0 + jax.lax.erf(h * 0.7071067811865476))
    contrib = g * act

    @pl.when(n == 0)
    def _():
        out_ref[...] = contrib

    @pl.when(n > 0)
    def _():
        out_ref[...] += contrib


def kernel(x, ctx_mod, context, W, b):
    N, B, L = x.shape
    n_ctx = context.shape[0]

    cn16, m, w16 = pl.pallas_call(
        _pre_kernel,
        out_shape=[
            jax.ShapeDtypeStruct((n_ctx, L), jnp.bfloat16),
            jax.ShapeDtypeStruct((1, n_ctx), jnp.float32),
            jax.ShapeDtypeStruct((L, L), jnp.bfloat16),
        ],
    )(context, ctx_mod, W)

    b2 = b.reshape(1, L)

    tb = 512 if B % 512 == 0 else B
    nb = B // tb
    out = pl.pallas_call(
        functools.partial(_main_kernel, n_total=N),
        grid=(nb, N),
        in_specs=[
            pl.BlockSpec((1, tb, L), lambda bi, n: (n, bi, 0)),
            pl.BlockSpec((n_ctx, L), lambda bi, n: (0, 0)),
            pl.BlockSpec((1, n_ctx), lambda bi, n: (0, 0)),
            pl.BlockSpec((L, L), lambda bi, n: (0, 0)),
            pl.BlockSpec((1, L), lambda bi, n: (0, 0)),
        ],
        out_specs=pl.BlockSpec((tb, L), lambda bi, n: (bi, 0)),
        out_shape=jax.ShapeDtypeStruct((B, L), jnp.float32),
    )(x, cn16, m, w16, b2)
    return out


# speedup vs baseline: 3.9002x; 1.0491x over previous
import functools

import jax
import jax.numpy as jnp
from jax.experimental import pallas as pl

_NT = (((1,), (1,)), ((), ()))


def _pre_kernel(ctx_ref, cm_ref, w_ref, cn_ref, m_ref, w16_ref):
    c = ctx_ref[...]
    norms = jnp.sqrt(jnp.sum(c * c, axis=1, keepdims=True))
    cn_ref[...] = (c / jnp.clip(norms, 1e-8, None)).astype(jnp.bfloat16)
    seg = jax.lax.dot_general(cm_ref[...], c, _NT,
                              preferred_element_type=jnp.float32)
    m_ref[...] = jnp.max(seg, axis=0, keepdims=True)
    w16_ref[...] = w_ref[...].astype(jnp.bfloat16)


def _main_kernel(x_ref, cn_ref, m_ref, w16_ref, b_ref, out_ref, *, n_total):
    n = pl.program_id(1)
    xb16 = x_ref[0].astype(jnp.bfloat16)
    s = jax.lax.dot_general(xb16, cn_ref[...], _NT,
                            preferred_element_type=jnp.float32)
    rowmax = jnp.max(s, axis=1, keepdims=True)
    mval = jnp.max(jnp.where(s == rowmax, m_ref[...], -jnp.inf),
                   axis=1, keepdims=True)
    act = jax.nn.sigmoid(mval) * (0.5 / n_total)
    h = jax.lax.dot_general(xb16, w16_ref[...], _NT,
                            preferred_element_type=jnp.float32) + b_ref[...]
    g = h * (1.0 + jax.lax.erf(h * 0.7071067811865476))
    contrib = g * act

    @pl.when(n == 0)
    def _():
        out_ref[...] = contrib

    @pl.when(n > 0)
    def _():
        out_ref[...] += contrib


def kernel(x, ctx_mod, context, W, b):
    N, B, L = x.shape
    n_ctx = context.shape[0]

    cn16, m, w16 = pl.pallas_call(
        _pre_kernel,
        out_shape=[
            jax.ShapeDtypeStruct((n_ctx, L), jnp.bfloat16),
            jax.ShapeDtypeStruct((1, n_ctx), jnp.float32),
            jax.ShapeDtypeStruct((L, L), jnp.bfloat16),
        ],
    )(context, ctx_mod, W)

    b2 = b.reshape(1, L)

    tb = 1024 if B % 1024 == 0 else B
    nb = B // tb
    out = pl.pallas_call(
        functools.partial(_main_kernel, n_total=N),
        grid=(nb, N),
        in_specs=[
            pl.BlockSpec((1, tb, L), lambda bi, n: (n, bi, 0)),
            pl.BlockSpec((n_ctx, L), lambda bi, n: (0, 0)),
            pl.BlockSpec((1, n_ctx), lambda bi, n: (0, 0)),
            pl.BlockSpec((L, L), lambda bi, n: (0, 0)),
            pl.BlockSpec((1, L), lambda bi, n: (0, 0)),
        ],
        out_specs=pl.BlockSpec((tb, L), lambda bi, n: (bi, 0)),
        out_shape=jax.ShapeDtypeStruct((B, L), jnp.float32),
    )(x, cn16, m, w16, b2)
    return out
